# baseline (speedup 1.0000x reference)
import math

import jax
import jax.numpy as jnp
from jax.experimental import pallas as pl
from jax.experimental.pallas import tpu as pltpu

B = 16
L = 576
SIZE = 24
D = 768
NUM_PARTS = 4
PARTS_DIM = D // NUM_PARTS
PART_CHANNELS = 32
GCN_DIM = 392
NUM_EDGES = NUM_PARTS * (NUM_PARTS - 1) * B
N_NODES = B * NUM_PARTS
BB = 2
HID = PARTS_DIM // 16

_INV_SQRT2 = 1.0 / math.sqrt(2.0)


def _gelu(t):
    return 0.5 * t * (1.0 + jax.lax.erf(t * _INV_SQRT2))


def _ln(t, g, b, eps=1e-6):
    m = jnp.mean(t, axis=-1, keepdims=True)
    v = jnp.mean((t - m) ** 2, axis=-1, keepdims=True)
    return (t - m) / jnp.sqrt(v + eps) * g + b


def _dot(a, b):
    return jnp.dot(a, b, preferred_element_type=jnp.float32)


def _cbam_body(x_ref, dm_ref, W1bd_ref, W2bd_ref, ws_ref, bs_ref, Wpbd_ref,
               bp_ref, x3_ref, m_ref, sh_ref):
    for bb in range(BB):
        xf = x_ref[bb]
        dmv = dm_ref[bb]
        avg_all = jnp.mean(xf, axis=0, keepdims=True)
        mx_all = jnp.max(xf, axis=0, keepdims=True)
        ha = jnp.maximum(_dot(avg_all, W1bd_ref[...]), 0.0)
        hm = jnp.maximum(_dot(mx_all, W1bd_ref[...]), 0.0)
        gate = jax.nn.sigmoid(_dot(ha, W2bd_ref[...])
                              + _dot(hm, W2bd_ref[...]))
        xg = xf * gate
        savg4 = jnp.concatenate(
            [jnp.mean(xg[:, p * PARTS_DIM:(p + 1) * PARTS_DIM],
                      axis=1, keepdims=True) for p in range(NUM_PARTS)],
            axis=1)
        smax4 = jnp.concatenate(
            [jnp.max(xg[:, p * PARTS_DIM:(p + 1) * PARTS_DIM],
                     axis=1, keepdims=True) for p in range(NUM_PARTS)],
            axis=1)
        sm4 = jax.nn.sigmoid(savg4 * ws_ref[0:1] + smax4 * ws_ref[1:2]
                             + dmv * ws_ref[2:3] + bs_ref[...])
        sm_exp = jnp.concatenate(
            [jnp.broadcast_to(sm4[:, p:p + 1], (L, PARTS_DIM))
             for p in range(NUM_PARTS)], axis=1)
        xs = xg * sm_exp
        xo = _dot(xs, Wpbd_ref[...]) + bp_ref[...]
        xoT = xo.T
        for p in range(NUM_PARTS):
            x3_ref[:, bb * NUM_PARTS + p, :] = (
                xoT[p * PART_CHANNELS:(p + 1) * PART_CHANNELS])
        m_ref[bb * NUM_PARTS:(bb + 1) * NUM_PARTS] = sm4.T
        sh_ref[bb] = avg_all


def _build_A(ei_ref, eit_ref, ews_ref, ewst_ref, blk):
    ew_row = jax.nn.sigmoid(ews_ref[blk:blk + 1, :])
    ew_col = jax.nn.sigmoid(ewst_ref[:, blk:blk + 1])
    dst_row = ei_ref[1:2, :]
    src_col = eit_ref[:, 0:1]
    dst_col = eit_ref[:, 1:2]
    row_ids = jax.lax.broadcasted_iota(jnp.int32, (N_NODES, NUM_EDGES), 0)
    col_ids = jax.lax.broadcasted_iota(jnp.int32, (NUM_EDGES, N_NODES), 1)
    Mdst = jnp.where(row_ids == dst_row, 1.0, 0.0)
    MdstT = jnp.where(col_ids == dst_col, 1.0, 0.0)
    Msrc = jnp.where(col_ids == src_col, 1.0, 0.0)
    deg_col = _dot(Mdst, ew_col) + 1.0
    deg_row = _dot(ew_row, MdstT) + 1.0
    dis_col = jnp.where(deg_col > 0,
                        jax.lax.rsqrt(jnp.maximum(deg_col, 1e-12)), 0.0)
    dis_row = jnp.where(deg_row > 0,
                        jax.lax.rsqrt(jnp.maximum(deg_row, 1e-12)), 0.0)
    W_raw = _dot(Mdst * ew_row, Msrc)
    ir = jax.lax.broadcasted_iota(jnp.int32, (N_NODES, N_NODES), 0)
    ic = jax.lax.broadcasted_iota(jnp.int32, (N_NODES, N_NODES), 1)
    eye = jnp.where(ir == ic, 1.0, 0.0)
    return dis_col * (W_raw + eye) * dis_row


def _fc_gcn_body(x3_ref, fw_ref, fb_ref, ei_ref, eit_ref, ews_ref, ewst_ref,
                 sh_ref,
                 W1a_ref, b1a_ref, g1a_ref, be1a_ref,
                 W2a_ref, b2a_ref, g2a_ref, be2a_ref,
                 W1b_ref, b1b_ref, g1b_ref, be1b_ref,
                 W2b_ref, b2b_ref, g2b_ref, be2b_ref,
                 Wd_ref, gd_ref, bd_ref,
                 out_ref, acc_ref):
    k = pl.program_id(0)

    @pl.when(k == 0)
    def _():
        acc_ref[...] = jnp.zeros_like(acc_ref)

    acc_ref[...] += _dot(x3_ref[0], fw_ref[...])

    @pl.when(k == PART_CHANNELS - 1)
    def _():
        y = acc_ref[...] + fb_ref[...]
        A1 = _build_A(ei_ref, eit_ref, ews_ref, ewst_ref, 0)
        A2 = _build_A(ei_ref, eit_ref, ews_ref, ewst_ref, 1)

        h = _dot(A1, _dot(y, W1a_ref[...])) + b1a_ref[...]
        h = _gelu(_ln(h, g1a_ref[...], be1a_ref[...]))
        h = _dot(A1, _dot(h, W2a_ref[...])) + b2a_ref[...]
        y1 = _gelu(_ln(h, g2a_ref[...], be2a_ref[...]) + y)

        h = _dot(A2, _dot(y1, W1b_ref[...])) + b1b_ref[...]
        h = _gelu(_ln(h, g1b_ref[...], be1b_ref[...]))
        h = _dot(A2, _dot(h, W2b_ref[...])) + b2b_ref[...]
        h = _ln(h, g2b_ref[...], be2b_ref[...])
        sc = _ln(_dot(y1, Wd_ref[...]), gd_ref[...], bd_ref[...])
        y2 = _gelu(h + sc)

        pr = jax.lax.broadcasted_iota(jnp.int32, (B, N_NODES), 0)
        pc = jax.lax.broadcasted_iota(jnp.int32, (B, N_NODES), 1)
        pool = jnp.where(pc // NUM_PARTS == pr, 0.25, 0.0)
        out_ref[...] = _dot(pool, y2) + sh_ref[...]


@jax.jit
def kernel(decision_masks, x, params, edge_index):
    cb = params['cbam']
    W1bd = jax.scipy.linalg.block_diag(*[c['W1'] for c in cb])
    W2bd = jax.scipy.linalg.block_diag(*[c['W2'] for c in cb])
    Wpbd = jax.scipy.linalg.block_diag(*[c['Wp'] for c in cb])
    bp_row = jnp.concatenate([c['bp'] for c in cb]).reshape(1, NUM_PARTS
                                                            * PART_CHANNELS)
    ws_cols = jnp.stack([c['Ws'] for c in cb], axis=1)
    bs_row = jnp.stack([c['bs'] for c in cb]).reshape(1, NUM_PARTS)

    x3, masks_t, short = pl.pallas_call(
        _cbam_body,
        grid=(B // BB,),
        in_specs=[
            pl.BlockSpec((BB, L, D), lambda b: (b, 0, 0)),
            pl.BlockSpec((BB, L, 1), lambda b: (b, 0, 0)),
            pl.BlockSpec((D, NUM_PARTS * HID), lambda b: (0, 0)),
            pl.BlockSpec((NUM_PARTS * HID, D), lambda b: (0, 0)),
            pl.BlockSpec((3, NUM_PARTS), lambda b: (0, 0)),
            pl.BlockSpec((1, NUM_PARTS), lambda b: (0, 0)),
            pl.BlockSpec((D, NUM_PARTS * PART_CHANNELS), lambda b: (0, 0)),
            pl.BlockSpec((1, NUM_PARTS * PART_CHANNELS), lambda b: (0, 0)),
        ],
        out_specs=[
            pl.BlockSpec((PART_CHANNELS, BB * NUM_PARTS, L),
                         lambda b: (0, b, 0)),
            pl.BlockSpec((BB * NUM_PARTS, L), lambda b: (b, 0)),
            pl.BlockSpec((BB, 1, D), lambda b: (b, 0, 0)),
        ],
        out_shape=[
            jax.ShapeDtypeStruct((PART_CHANNELS, N_NODES, L), jnp.float32),
            jax.ShapeDtypeStruct((N_NODES, L), jnp.float32),
            jax.ShapeDtypeStruct((B, 1, D), jnp.float32),
        ],
    )(x, decision_masks, W1bd, W2bd, ws_cols, bs_row, Wpbd, bp_row)

    short = short.reshape(B, D)
    parts_masks = masks_t.reshape(B, NUM_PARTS, SIZE, SIZE)

    blocks = params['blocks']
    ews = jnp.stack([bp['edge_weight'] for bp in blocks])
    ewst = ews.T
    ei = edge_index.astype(jnp.int32)
    eit = ei.T
    b0, b1 = blocks

    full = lambda s: pl.BlockSpec(s, lambda k: tuple(0 for _ in s))
    r2 = lambda a: a.reshape(1, -1)

    out = pl.pallas_call(
        _fc_gcn_body,
        grid=(PART_CHANNELS,),
        in_specs=[
            pl.BlockSpec((1, N_NODES, L), lambda k: (k, 0, 0)),
            pl.BlockSpec((L, GCN_DIM), lambda k: (k, 0)),
            full((1, GCN_DIM)),
            full((2, NUM_EDGES)),
            full((NUM_EDGES, 2)),
            full((2, NUM_EDGES)),
            full((NUM_EDGES, 2)),
            full((B, D)),
            full(b0['W1'].shape), full((1, b0['b1'].shape[0])),
            full((1, b0['g1'].shape[0])), full((1, b0['be1'].shape[0])),
            full(b0['W2'].shape), full((1, b0['b2'].shape[0])),
            full((1, b0['g2'].shape[0])), full((1, b0['be2'].shape[0])),
            full(b1['W1'].shape), full((1, b1['b1'].shape[0])),
            full((1, b1['g1'].shape[0])), full((1, b1['be1'].shape[0])),
            full(b1['W2'].shape), full((1, b1['b2'].shape[0])),
            full((1, b1['g2'].shape[0])), full((1, b1['be2'].shape[0])),
            full(b1['Wd'].shape), full((1, b1['gd'].shape[0])),
            full((1, b1['bd'].shape[0])),
        ],
        out_specs=pl.BlockSpec((B, D), lambda k: (0, 0)),
        out_shape=jax.ShapeDtypeStruct((B, D), jnp.float32),
        scratch_shapes=[pltpu.VMEM((N_NODES, GCN_DIM), jnp.float32)],
    )(x3, params['fc_W'], r2(params['fc_b']), ei, eit, ews, ewst, short,
      b0['W1'], r2(b0['b1']), r2(b0['g1']), r2(b0['be1']),
      b0['W2'], r2(b0['b2']), r2(b0['g2']), r2(b0['be2']),
      b1['W1'], r2(b1['b1']), r2(b1['g1']), r2(b1['be1']),
      b1['W2'], r2(b1['b2']), r2(b1['g2']), r2(b1['be2']),
      b1['Wd'], r2(b1['gd']), r2(b1['bd']))

    return out, parts_masks


# speedup vs baseline: 1.5796x; 1.5796x over previous
import math

import jax
import jax.numpy as jnp
from jax.experimental import pallas as pl
from jax.experimental.pallas import tpu as pltpu

B = 16
L = 576
SIZE = 24
D = 768
NUM_PARTS = 4
PARTS_DIM = D // NUM_PARTS
PART_CHANNELS = 32
GCN_DIM = 392
NUM_EDGES = NUM_PARTS * (NUM_PARTS - 1) * B
N_NODES = B * NUM_PARTS
BB = 2
HID = PARTS_DIM // 16
CPS = 4
NKB = PART_CHANNELS // CPS

_INV_SQRT2 = 1.0 / math.sqrt(2.0)


def _gelu(t):
    return 0.5 * t * (1.0 + jax.lax.erf(t * _INV_SQRT2))


def _ln(t, g, b, eps=1e-6):
    m = jnp.mean(t, axis=-1, keepdims=True)
    v = jnp.mean((t - m) ** 2, axis=-1, keepdims=True)
    return (t - m) / jnp.sqrt(v + eps) * g + b


def _dot(a, b):
    return jnp.dot(a, b, preferred_element_type=jnp.float32)


def _cbam_body(x_ref, dm_ref, W1bd_ref, W2bd_ref, ws_ref, bs_ref, Wpbd_ref,
               bp_ref, x3_ref, m_ref, sh_ref):
    for bb in range(BB):
        xf = x_ref[bb]
        dmv = dm_ref[bb]
        avg_all = jnp.mean(xf, axis=0, keepdims=True)
        mx_all = jnp.max(xf, axis=0, keepdims=True)
        ha = jnp.maximum(_dot(avg_all, W1bd_ref[...]), 0.0)
        hm = jnp.maximum(_dot(mx_all, W1bd_ref[...]), 0.0)
        gate = jax.nn.sigmoid(_dot(ha, W2bd_ref[...])
                              + _dot(hm, W2bd_ref[...]))
        xg = xf * gate
        savg4 = jnp.concatenate(
            [jnp.mean(xg[:, p * PARTS_DIM:(p + 1) * PARTS_DIM],
                      axis=1, keepdims=True) for p in range(NUM_PARTS)],
            axis=1)
        smax4 = jnp.concatenate(
            [jnp.max(xg[:, p * PARTS_DIM:(p + 1) * PARTS_DIM],
                     axis=1, keepdims=True) for p in range(NUM_PARTS)],
            axis=1)
        sm4 = jax.nn.sigmoid(savg4 * ws_ref[0:1] + smax4 * ws_ref[1:2]
                             + dmv * ws_ref[2:3] + bs_ref[...])
        sm_exp = jnp.concatenate(
            [jnp.broadcast_to(sm4[:, p:p + 1], (L, PARTS_DIM))
             for p in range(NUM_PARTS)], axis=1)
        xs = xg * sm_exp
        xo = _dot(xs, Wpbd_ref[...]) + bp_ref[...]
        xoT = xo.T
        for p in range(NUM_PARTS):
            x3_ref[:, bb * NUM_PARTS + p, :] = (
                xoT[p * PART_CHANNELS:(p + 1) * PART_CHANNELS])
        m_ref[bb * NUM_PARTS:(bb + 1) * NUM_PARTS] = sm4.T
        sh_ref[bb] = avg_all


def _build_A(ei_ref, eit_ref, ews_ref, ewst_ref, blk):
    ew_row = jax.nn.sigmoid(ews_ref[blk:blk + 1, :])
    ew_col = jax.nn.sigmoid(ewst_ref[:, blk:blk + 1])
    dst_row = ei_ref[1:2, :]
    src_col = eit_ref[:, 0:1]
    dst_col = eit_ref[:, 1:2]
    row_ids = jax.lax.broadcasted_iota(jnp.int32, (N_NODES, NUM_EDGES), 0)
    col_ids = jax.lax.broadcasted_iota(jnp.int32, (NUM_EDGES, N_NODES), 1)
    Mdst = jnp.where(row_ids == dst_row, 1.0, 0.0)
    MdstT = jnp.where(col_ids == dst_col, 1.0, 0.0)
    Msrc = jnp.where(col_ids == src_col, 1.0, 0.0)
    deg_col = _dot(Mdst, ew_col) + 1.0
    deg_row = _dot(ew_row, MdstT) + 1.0
    dis_col = jnp.where(deg_col > 0,
                        jax.lax.rsqrt(jnp.maximum(deg_col, 1e-12)), 0.0)
    dis_row = jnp.where(deg_row > 0,
                        jax.lax.rsqrt(jnp.maximum(deg_row, 1e-12)), 0.0)
    W_raw = _dot(Mdst * ew_row, Msrc)
    ir = jax.lax.broadcasted_iota(jnp.int32, (N_NODES, N_NODES), 0)
    ic = jax.lax.broadcasted_iota(jnp.int32, (N_NODES, N_NODES), 1)
    eye = jnp.where(ir == ic, 1.0, 0.0)
    return dis_col * (W_raw + eye) * dis_row


def _fc_gcn_body(x3_ref, fw_ref, fb_ref, ei_ref, eit_ref, ews_ref, ewst_ref,
                 sh_ref,
                 W1a_ref, b1a_ref, g1a_ref, be1a_ref,
                 W2a_ref, b2a_ref, g2a_ref, be2a_ref,
                 W1b_ref, b1b_ref, g1b_ref, be1b_ref,
                 W2b_ref, b2b_ref, g2b_ref, be2b_ref,
                 Wd_ref, gd_ref, bd_ref,
                 out_ref, acc_ref):
    k = pl.program_id(0)

    @pl.when(k == 0)
    def _():
        acc_ref[...] = jnp.zeros_like(acc_ref)

    xgT = jnp.concatenate([x3_ref[j].T for j in range(CPS)], axis=0)
    acc_ref[...] += _dot(fw_ref[...], xgT)

    @pl.when(k == NKB - 1)
    def _():
        y = acc_ref[...].T + fb_ref[...]
        A1 = _build_A(ei_ref, eit_ref, ews_ref, ewst_ref, 0)
        A2 = _build_A(ei_ref, eit_ref, ews_ref, ewst_ref, 1)

        h = _dot(A1, _dot(y, W1a_ref[...])) + b1a_ref[...]
        h = _gelu(_ln(h, g1a_ref[...], be1a_ref[...]))
        h = _dot(A1, _dot(h, W2a_ref[...])) + b2a_ref[...]
        y1 = _gelu(_ln(h, g2a_ref[...], be2a_ref[...]) + y)

        h = _dot(A2, _dot(y1, W1b_ref[...])) + b1b_ref[...]
        h = _gelu(_ln(h, g1b_ref[...], be1b_ref[...]))
        h = _dot(A2, _dot(h, W2b_ref[...])) + b2b_ref[...]
        h = _ln(h, g2b_ref[...], be2b_ref[...])
        sc = _ln(_dot(y1, Wd_ref[...]), gd_ref[...], bd_ref[...])
        y2 = _gelu(h + sc)

        pr = jax.lax.broadcasted_iota(jnp.int32, (B, N_NODES), 0)
        pc = jax.lax.broadcasted_iota(jnp.int32, (B, N_NODES), 1)
        pool = jnp.where(pc // NUM_PARTS == pr, 0.25, 0.0)
        out_ref[...] = _dot(pool, y2) + sh_ref[...]


@jax.jit
def kernel(decision_masks, x, params, edge_index):
    cb = params['cbam']
    W1bd = jax.scipy.linalg.block_diag(*[c['W1'] for c in cb])
    W2bd = jax.scipy.linalg.block_diag(*[c['W2'] for c in cb])
    Wpbd = jax.scipy.linalg.block_diag(*[c['Wp'] for c in cb])
    bp_row = jnp.concatenate([c['bp'] for c in cb]).reshape(1, NUM_PARTS
                                                            * PART_CHANNELS)
    ws_cols = jnp.stack([c['Ws'] for c in cb], axis=1)
    bs_row = jnp.stack([c['bs'] for c in cb]).reshape(1, NUM_PARTS)

    x3, masks_t, short = pl.pallas_call(
        _cbam_body,
        grid=(B // BB,),
        in_specs=[
            pl.BlockSpec((BB, L, D), lambda b: (b, 0, 0)),
            pl.BlockSpec((BB, L, 1), lambda b: (b, 0, 0)),
            pl.BlockSpec((D, NUM_PARTS * HID), lambda b: (0, 0)),
            pl.BlockSpec((NUM_PARTS * HID, D), lambda b: (0, 0)),
            pl.BlockSpec((3, NUM_PARTS), lambda b: (0, 0)),
            pl.BlockSpec((1, NUM_PARTS), lambda b: (0, 0)),
            pl.BlockSpec((D, NUM_PARTS * PART_CHANNELS), lambda b: (0, 0)),
            pl.BlockSpec((1, NUM_PARTS * PART_CHANNELS), lambda b: (0, 0)),
        ],
        out_specs=[
            pl.BlockSpec((PART_CHANNELS, BB * NUM_PARTS, L),
                         lambda b: (0, b, 0)),
            pl.BlockSpec((BB * NUM_PARTS, L), lambda b: (b, 0)),
            pl.BlockSpec((BB, 1, D), lambda b: (b, 0, 0)),
        ],
        out_shape=[
            jax.ShapeDtypeStruct((PART_CHANNELS, N_NODES, L), jnp.float32),
            jax.ShapeDtypeStruct((N_NODES, L), jnp.float32),
            jax.ShapeDtypeStruct((B, 1, D), jnp.float32),
        ],
    )(x, decision_masks, W1bd, W2bd, ws_cols, bs_row, Wpbd, bp_row)

    short = short.reshape(B, D)
    parts_masks = masks_t.reshape(B, NUM_PARTS, SIZE, SIZE)

    blocks = params['blocks']
    ews = jnp.stack([bp['edge_weight'] for bp in blocks])
    ewst = ews.T
    ei = edge_index.astype(jnp.int32)
    eit = ei.T
    b0, b1 = blocks

    full = lambda s: pl.BlockSpec(s, lambda k: tuple(0 for _ in s))
    r2 = lambda a: a.reshape(1, -1)

    out = pl.pallas_call(
        _fc_gcn_body,
        grid=(NKB,),
        in_specs=[
            pl.BlockSpec((CPS, N_NODES, L), lambda k: (k, 0, 0)),
            pl.BlockSpec((GCN_DIM, CPS * L), lambda k: (0, k)),
            full((1, GCN_DIM)),
            full((2, NUM_EDGES)),
            full((NUM_EDGES, 2)),
            full((2, NUM_EDGES)),
            full((NUM_EDGES, 2)),
            full((B, D)),
            full(b0['W1'].shape), full((1, b0['b1'].shape[0])),
            full((1, b0['g1'].shape[0])), full((1, b0['be1'].shape[0])),
            full(b0['W2'].shape), full((1, b0['b2'].shape[0])),
            full((1, b0['g2'].shape[0])), full((1, b0['be2'].shape[0])),
            full(b1['W1'].shape), full((1, b1['b1'].shape[0])),
            full((1, b1['g1'].shape[0])), full((1, b1['be1'].shape[0])),
            full(b1['W2'].shape), full((1, b1['b2'].shape[0])),
            full((1, b1['g2'].shape[0])), full((1, b1['be2'].shape[0])),
            full(b1['Wd'].shape), full((1, b1['gd'].shape[0])),
            full((1, b1['bd'].shape[0])),
        ],
        out_specs=pl.BlockSpec((B, D), lambda k: (0, 0)),
        out_shape=jax.ShapeDtypeStruct((B, D), jnp.float32),
        scratch_shapes=[pltpu.VMEM((GCN_DIM, N_NODES), jnp.float32)],
    )(x3, params['fc_W'].T, r2(params['fc_b']), ei, eit, ews, ewst, short,
      b0['W1'], r2(b0['b1']), r2(b0['g1']), r2(b0['be1']),
      b0['W2'], r2(b0['b2']), r2(b0['g2']), r2(b0['be2']),
      b1['W1'], r2(b1['b1']), r2(b1['g1']), r2(b1['be1']),
      b1['W2'], r2(b1['b2']), r2(b1['g2']), r2(b1['be2']),
      b1['Wd'], r2(b1['gd']), r2(b1['bd']))

    return out, parts_masks


# speedup vs baseline: 1.8922x; 1.1979x over previous
import math

import jax
import jax.numpy as jnp
from jax.experimental import pallas as pl
from jax.experimental.pallas import tpu as pltpu

B = 16
L = 576
SIZE = 24
D = 768
NUM_PARTS = 4
PARTS_DIM = D // NUM_PARTS
PART_CHANNELS = 32
GCN_DIM = 392
NUM_EDGES = NUM_PARTS * (NUM_PARTS - 1) * B
N_NODES = B * NUM_PARTS
BB = 2
HID = PARTS_DIM // 16
CPS = 4
NKB = PART_CHANNELS // CPS

_INV_SQRT2 = 1.0 / math.sqrt(2.0)


def _gelu(t):
    return 0.5 * t * (1.0 + jax.lax.erf(t * _INV_SQRT2))


def _ln(t, g, b, eps=1e-6):
    m = jnp.mean(t, axis=-1, keepdims=True)
    v = jnp.mean((t - m) ** 2, axis=-1, keepdims=True)
    return (t - m) / jnp.sqrt(v + eps) * g + b


def _dot(a, b):
    return jnp.dot(a, b, preferred_element_type=jnp.float32)


def _cbam_body(x_ref, dm_ref, W1bd_ref, W2bd_ref, ws_ref, bs_ref, Wpe_ref,
               e32_ref, bp_ref, x3_ref, m_ref, sh_ref):
    for bb in range(BB):
        xf = x_ref[bb]
        dmv = dm_ref[bb]
        avg_all = jnp.mean(xf, axis=0, keepdims=True)
        mx_all = jnp.max(xf, axis=0, keepdims=True)
        ha = jnp.maximum(_dot(avg_all, W1bd_ref[...]), 0.0)
        hm = jnp.maximum(_dot(mx_all, W1bd_ref[...]), 0.0)
        gate = jax.nn.sigmoid(_dot(ha, W2bd_ref[...])
                              + _dot(hm, W2bd_ref[...]))
        xg = xf * gate
        z = _dot(xg, Wpe_ref[...])
        savg4 = z[:, NUM_PARTS * PART_CHANNELS:]
        smax4 = jnp.concatenate(
            [jnp.max(xg[:, p * PARTS_DIM:(p + 1) * PARTS_DIM],
                     axis=1, keepdims=True) for p in range(NUM_PARTS)],
            axis=1)
        sm4 = jax.nn.sigmoid(savg4 * ws_ref[0:1] + smax4 * ws_ref[1:2]
                             + dmv * ws_ref[2:3] + bs_ref[...])
        sm128 = _dot(sm4, e32_ref[...])
        xo = z[:, :NUM_PARTS * PART_CHANNELS] * sm128 + bp_ref[...]
        xoT = xo.T
        for p in range(NUM_PARTS):
            x3_ref[:, bb * NUM_PARTS + p, :] = (
                xoT[p * PART_CHANNELS:(p + 1) * PART_CHANNELS])
        m_ref[bb * NUM_PARTS:(bb + 1) * NUM_PARTS] = sm4.T
        sh_ref[bb] = avg_all


def _build_A(ei_ref, eit_ref, ews_ref, ewst_ref, blk):
    ew_row = jax.nn.sigmoid(ews_ref[blk:blk + 1, :])
    ew_col = jax.nn.sigmoid(ewst_ref[:, blk:blk + 1])
    dst_row = ei_ref[1:2, :]
    src_col = eit_ref[:, 0:1]
    dst_col = eit_ref[:, 1:2]
    row_ids = jax.lax.broadcasted_iota(jnp.int32, (N_NODES, NUM_EDGES), 0)
    col_ids = jax.lax.broadcasted_iota(jnp.int32, (NUM_EDGES, N_NODES), 1)
    Mdst = jnp.where(row_ids == dst_row, 1.0, 0.0)
    MdstT = jnp.where(col_ids == dst_col, 1.0, 0.0)
    Msrc = jnp.where(col_ids == src_col, 1.0, 0.0)
    deg_col = _dot(Mdst, ew_col) + 1.0
    deg_row = _dot(ew_row, MdstT) + 1.0
    dis_col = jnp.where(deg_col > 0,
                        jax.lax.rsqrt(jnp.maximum(deg_col, 1e-12)), 0.0)
    dis_row = jnp.where(deg_row > 0,
                        jax.lax.rsqrt(jnp.maximum(deg_row, 1e-12)), 0.0)
    W_raw = _dot(Mdst * ew_row, Msrc)
    ir = jax.lax.broadcasted_iota(jnp.int32, (N_NODES, N_NODES), 0)
    ic = jax.lax.broadcasted_iota(jnp.int32, (N_NODES, N_NODES), 1)
    eye = jnp.where(ir == ic, 1.0, 0.0)
    return dis_col * (W_raw + eye) * dis_row


def _fc_gcn_body(x3_ref, fw_ref, fb_ref, ei_ref, eit_ref, ews_ref, ewst_ref,
                 sh_ref,
                 W1a_ref, b1a_ref, g1a_ref, be1a_ref,
                 W2a_ref, b2a_ref, g2a_ref, be2a_ref,
                 W1b_ref, b1b_ref, g1b_ref, be1b_ref,
                 W2b_ref, b2b_ref, g2b_ref, be2b_ref,
                 Wd_ref, gd_ref, bd_ref,
                 out_ref, acc_ref):
    k = pl.program_id(0)

    @pl.when(k == 0)
    def _():
        acc_ref[...] = jnp.zeros_like(acc_ref)

    xgT = jnp.concatenate([x3_ref[j].T for j in range(CPS)], axis=0)
    acc_ref[...] += _dot(fw_ref[...], xgT)

    @pl.when(k == NKB - 1)
    def _():
        y = acc_ref[...].T + fb_ref[...]
        A1 = _build_A(ei_ref, eit_ref, ews_ref, ewst_ref, 0)
        A2 = _build_A(ei_ref, eit_ref, ews_ref, ewst_ref, 1)

        h = _dot(A1, _dot(y, W1a_ref[...])) + b1a_ref[...]
        h = _gelu(_ln(h, g1a_ref[...], be1a_ref[...]))
        h = _dot(A1, _dot(h, W2a_ref[...])) + b2a_ref[...]
        y1 = _gelu(_ln(h, g2a_ref[...], be2a_ref[...]) + y)

        h = _dot(A2, _dot(y1, W1b_ref[...])) + b1b_ref[...]
        h = _gelu(_ln(h, g1b_ref[...], be1b_ref[...]))
        h = _dot(A2, _dot(h, W2b_ref[...])) + b2b_ref[...]
        h = _ln(h, g2b_ref[...], be2b_ref[...])
        sc = _ln(_dot(y1, Wd_ref[...]), gd_ref[...], bd_ref[...])
        y2 = _gelu(h + sc)

        pr = jax.lax.broadcasted_iota(jnp.int32, (B, N_NODES), 0)
        pc = jax.lax.broadcasted_iota(jnp.int32, (B, N_NODES), 1)
        pool = jnp.where(pc // NUM_PARTS == pr, 0.25, 0.0)
        out_ref[...] = _dot(pool, y2) + sh_ref[...]


@jax.jit
def kernel(decision_masks, x, params, edge_index):
    cb = params['cbam']
    W1bd = jax.scipy.linalg.block_diag(*[c['W1'] for c in cb])
    W2bd = jax.scipy.linalg.block_diag(*[c['W2'] for c in cb])
    Wpbd = jax.scipy.linalg.block_diag(*[c['Wp'] for c in cb])
    part_of_d = jnp.arange(D, dtype=jnp.int32) // PARTS_DIM
    bandmask = jnp.where(part_of_d[:, None]
                         == jnp.arange(NUM_PARTS, dtype=jnp.int32)[None, :],
                         1.0 / PARTS_DIM, 0.0)
    Wpe = jnp.concatenate([Wpbd, bandmask], axis=1)
    part_of_c = jnp.arange(NUM_PARTS * PART_CHANNELS,
                           dtype=jnp.int32) // PART_CHANNELS
    e32 = jnp.where(jnp.arange(NUM_PARTS, dtype=jnp.int32)[:, None]
                    == part_of_c[None, :], 1.0, 0.0)
    bp_row = jnp.concatenate([c['bp'] for c in cb]).reshape(1, NUM_PARTS
                                                            * PART_CHANNELS)
    ws_cols = jnp.stack([c['Ws'] for c in cb], axis=1)
    bs_row = jnp.stack([c['bs'] for c in cb]).reshape(1, NUM_PARTS)

    x3, masks_t, short = pl.pallas_call(
        _cbam_body,
        grid=(B // BB,),
        in_specs=[
            pl.BlockSpec((BB, L, D), lambda b: (b, 0, 0)),
            pl.BlockSpec((BB, L, 1), lambda b: (b, 0, 0)),
            pl.BlockSpec((D, NUM_PARTS * HID), lambda b: (0, 0)),
            pl.BlockSpec((NUM_PARTS * HID, D), lambda b: (0, 0)),
            pl.BlockSpec((3, NUM_PARTS), lambda b: (0, 0)),
            pl.BlockSpec((1, NUM_PARTS), lambda b: (0, 0)),
            pl.BlockSpec((D, NUM_PARTS * PART_CHANNELS + NUM_PARTS),
                         lambda b: (0, 0)),
            pl.BlockSpec((NUM_PARTS, NUM_PARTS * PART_CHANNELS),
                         lambda b: (0, 0)),
            pl.BlockSpec((1, NUM_PARTS * PART_CHANNELS), lambda b: (0, 0)),
        ],
        out_specs=[
            pl.BlockSpec((PART_CHANNELS, BB * NUM_PARTS, L),
                         lambda b: (0, b, 0)),
            pl.BlockSpec((BB * NUM_PARTS, L), lambda b: (b, 0)),
            pl.BlockSpec((BB, 1, D), lambda b: (b, 0, 0)),
        ],
        out_shape=[
            jax.ShapeDtypeStruct((PART_CHANNELS, N_NODES, L), jnp.float32),
            jax.ShapeDtypeStruct((N_NODES, L), jnp.float32),
            jax.ShapeDtypeStruct((B, 1, D), jnp.float32),
        ],
    )(x, decision_masks, W1bd, W2bd, ws_cols, bs_row, Wpe, e32, bp_row)

    short = short.reshape(B, D)
    parts_masks = masks_t.reshape(B, NUM_PARTS, SIZE, SIZE)

    blocks = params['blocks']
    ews = jnp.stack([bp['edge_weight'] for bp in blocks])
    ewst = ews.T
    ei = edge_index.astype(jnp.int32)
    eit = ei.T
    b0, b1 = blocks

    full = lambda s: pl.BlockSpec(s, lambda k: tuple(0 for _ in s))
    r2 = lambda a: a.reshape(1, -1)

    out = pl.pallas_call(
        _fc_gcn_body,
        grid=(NKB,),
        in_specs=[
            pl.BlockSpec((CPS, N_NODES, L), lambda k: (k, 0, 0)),
            pl.BlockSpec((GCN_DIM, CPS * L), lambda k: (0, k)),
            full((1, GCN_DIM)),
            full((2, NUM_EDGES)),
            full((NUM_EDGES, 2)),
            full((2, NUM_EDGES)),
            full((NUM_EDGES, 2)),
            full((B, D)),
            full(b0['W1'].shape), full((1, b0['b1'].shape[0])),
            full((1, b0['g1'].shape[0])), full((1, b0['be1'].shape[0])),
            full(b0['W2'].shape), full((1, b0['b2'].shape[0])),
            full((1, b0['g2'].shape[0])), full((1, b0['be2'].shape[0])),
            full(b1['W1'].shape), full((1, b1['b1'].shape[0])),
            full((1, b1['g1'].shape[0])), full((1, b1['be1'].shape[0])),
            full(b1['W2'].shape), full((1, b1['b2'].shape[0])),
            full((1, b1['g2'].shape[0])), full((1, b1['be2'].shape[0])),
            full(b1['Wd'].shape), full((1, b1['gd'].shape[0])),
            full((1, b1['bd'].shape[0])),
        ],
        out_specs=pl.BlockSpec((B, D), lambda k: (0, 0)),
        out_shape=jax.ShapeDtypeStruct((B, D), jnp.float32),
        scratch_shapes=[pltpu.VMEM((GCN_DIM, N_NODES), jnp.float32)],
    )(x3, params['fc_W'].T, r2(params['fc_b']), ei, eit, ews, ewst, short,
      b0['W1'], r2(b0['b1']), r2(b0['g1']), r2(b0['be1']),
      b0['W2'], r2(b0['b2']), r2(b0['g2']), r2(b0['be2']),
      b1['W1'], r2(b1['b1']), r2(b1['g1']), r2(b1['be1']),
      b1['W2'], r2(b1['b2']), r2(b1['g2']), r2(b1['be2']),
      b1['Wd'], r2(b1['gd']), r2(b1['bd']))

    return out, parts_masks
